# Initial kernel scaffold; baseline (speedup 1.0000x reference)
import jax
import jax.numpy as jnp
from jax.experimental import pallas as pl


def kernel(y, eval_gene_idx, train_highly_gene_idx, train_low_gene_idx, index, unnorm_index, thresh):
    raise NotImplementedError("write your pallas kernel here")


# speedup vs baseline: 1.6527x; 1.6527x over previous
import functools

import jax
import jax.numpy as jnp
from jax import lax
from jax.experimental import pallas as pl
from jax.experimental.pallas import tpu as pltpu
from jax.experimental.pallas import tpu_sc as plsc

_TEMPER = 0.5
_NC = 2
_NS = 16
_L = 16


@functools.lru_cache(maxsize=None)
def _build(n_total: int, n_hvg: int):
    nw = _NC * _NS
    ch = ((n_total + nw - 1) // nw + _L - 1) // _L * _L
    ch = max(ch, _L)
    assert ch % 8 == 0 and n_total % 8 == 0 and n_hvg % _L == 0
    nv = n_hvg // _L
    last_off = n_total - ch

    mesh = plsc.VectorSubcoreMesh(core_axis_name="c", subcore_axis_name="s")

    @functools.partial(
        pl.kernel,
        out_type=(
            jax.ShapeDtypeStruct((n_total,), jnp.float32),
            jax.ShapeDtypeStruct((_L,), jnp.int32),
        ),
        mesh=mesh,
        compiler_params=pltpu.CompilerParams(needs_layout_passes=False),
        scratch_types=[
            pltpu.VMEM((n_hvg,), jnp.int32),
            pltpu.VMEM((n_hvg,), jnp.float32),
            pltpu.VMEM((n_hvg,), jnp.int32),
            pltpu.VMEM((_L,), jnp.float32),
            pltpu.VMEM((ch,), jnp.float32),
            pltpu.VMEM((_L,), jnp.int32),
        ],
    )
    def sc_kernel(hvg_hbm, idx_hbm, unn_hbm, th_hbm, w_hbm, k_hbm,
                  hvg_v, idx_v, unn_v, th_v, wbuf, kbuf):
        cid = lax.axis_index("c")
        sid = lax.axis_index("s")
        wid = sid * _NC + cid
        off = jnp.minimum(wid * ch, last_off)

        pltpu.sync_copy(th_hbm, th_v)
        pltpu.sync_copy(hvg_hbm, hvg_v)
        pltpu.sync_copy(idx_hbm, idx_v)
        t = th_v[...]

        ones16 = jnp.full((_L,), 1.0, jnp.float32)

        def fill(j, carry):
            wbuf[pl.ds(j * _L, _L)] = ones16
            return carry

        lax.fori_loop(0, ch // _L, fill, 0)

        def scan(j, carry):
            g = hvg_v[pl.ds(j * _L, _L)]
            iv = idx_v[pl.ds(j * _L, _L)]
            m = 1.0 / (1.0 + jnp.exp((iv - t) * (1.0 / _TEMPER)))
            inr = (g >= off) & (g < off + ch)
            loc = jnp.clip(g - off, 0, ch - 1)
            plsc.store_scatter(wbuf, [loc], m, mask=inr)
            return carry

        lax.fori_loop(0, nv, scan, 0)
        pltpu.sync_copy(wbuf, w_hbm.at[pl.ds(off, ch)])

        @pl.when(wid == 0)
        def _():
            pltpu.sync_copy(unn_hbm, unn_v)
            lane = lax.iota(jnp.int32, _L)

            def amin(j, carry):
                best, bpos = carry
                iv = idx_v[pl.ds(j * _L, _L)]
                d = jnp.abs(iv - t)
                pos = j * _L + lane
                upd = d < best
                return (jnp.where(upd, d, best), jnp.where(upd, pos, bpos))

            best, bpos = lax.fori_loop(
                0, nv, amin,
                (jnp.full((_L,), jnp.inf, jnp.float32),
                 jnp.zeros((_L,), jnp.int32)))
            mn = jnp.min(best)
            pc = jnp.where(best == mn, bpos, jnp.int32(2 ** 30))
            p = jnp.min(pc)
            kbuf[...] = plsc.load_gather(unn_v, [jnp.full((_L,), p, jnp.int32)])
            pltpu.sync_copy(kbuf, k_hbm)

    return sc_kernel


def kernel(y, eval_gene_idx, train_highly_gene_idx, train_low_gene_idx,
           index, unnorm_index, thresh):
    n_total = (eval_gene_idx.shape[0] + train_highly_gene_idx.shape[0]
               + train_low_gene_idx.shape[0])
    n_hvg = train_highly_gene_idx.shape[0]
    th16 = jnp.full((_L,), thresh, dtype=jnp.float32)
    w, kv = _build(n_total, n_hvg)(
        train_highly_gene_idx, index, unnorm_index, th16)
    return (w, w, thresh, kv[0])


# speedup vs baseline: 3.1822x; 1.9254x over previous
import functools

import jax
import jax.numpy as jnp
from jax import lax
from jax.experimental import pallas as pl
from jax.experimental.pallas import tpu as pltpu
from jax.experimental.pallas import tpu_sc as plsc

_TEMPER = 0.5
_NC = 2
_NS = 16
_L = 16
_BIG = 2 ** 30


@functools.lru_cache(maxsize=None)
def _build(n_total: int, n_hvg: int):
    nw = _NC * _NS
    ch = ((n_total + nw - 1) // nw + _L - 1) // _L * _L
    ch = max(ch, _L)
    assert ch % 8 == 0 and n_total % 8 == 0 and n_hvg % _L == 0
    last_off = n_total - ch
    nvec = n_hvg // _L
    bs_iters = max(1, (nvec - 1).bit_length()) + 1
    av = -(-n_hvg // (_NS * _L))
    a_last = n_hvg - av * _L

    mesh = plsc.VectorSubcoreMesh(core_axis_name="c", subcore_axis_name="s")

    @functools.partial(
        pl.kernel,
        out_type=(
            jax.ShapeDtypeStruct((n_total,), jnp.float32),
            jax.ShapeDtypeStruct((_L,), jnp.int32),
        ),
        mesh=mesh,
        compiler_params=pltpu.CompilerParams(needs_layout_passes=False),
        scratch_types=[
            pltpu.VMEM((n_hvg,), jnp.int32),
            pltpu.VMEM((n_hvg,), jnp.float32),
            pltpu.VMEM((_L,), jnp.float32),
            pltpu.VMEM((ch,), jnp.float32),
            pltpu.VMEM((_L,), jnp.int32),
            pltpu.VMEM((_L,), jnp.int32),
            pltpu.VMEM((_L,), jnp.float32),
            pltpu.VMEM((_L,), jnp.int32),
            pltpu.VMEM((_NS * _L,), jnp.float32),
            pltpu.VMEM((_NS * _L,), jnp.int32),
            pltpu.VMEM_SHARED((_NS * _L,), jnp.float32),
            pltpu.VMEM_SHARED((_NS * _L,), jnp.int32),
            pltpu.SemaphoreType.DMA,
            pltpu.SemaphoreType.DMA,
            pltpu.SemaphoreType.DMA,
            pltpu.SemaphoreType.DMA,
        ],
    )
    def sc_kernel(hvg_hbm, idx_hbm, unn_hbm, th_hbm, w_hbm, k_hbm,
                  hvg_v, idx_v, th_v, wbuf, kbuf, pbuf, bd_v, bp_v,
                  red_d, red_p, sh_d, sh_p, sem_h, sem_i, sem_t, sem_k):
        cid = lax.axis_index("c")
        sid = lax.axis_index("s")
        wid = sid * _NC + cid
        off = jnp.minimum(wid * ch, last_off)

        cp_h = pltpu.async_copy(hvg_hbm, hvg_v, sem_h)
        cp_i = pltpu.async_copy(idx_hbm, idx_v, sem_i)
        cp_t = pltpu.async_copy(th_hbm, th_v, sem_t)

        ones16 = jnp.full((_L,), 1.0, jnp.float32)

        def fill(j, carry):
            wbuf[pl.ds(j * _L, _L)] = ones16
            return carry

        lax.fori_loop(0, ch // _L, fill, 0)
        cp_h.wait()
        cp_i.wait()
        cp_t.wait()
        t = th_v[...]

        def lower_bound_vec(target):
            def step(_, lohi):
                lo, hi = lohi
                mid = lax.div(lo + hi, jnp.int32(2))
                ld = jnp.minimum(mid, jnp.int32(nvec - 1))
                below = hvg_v[pl.ds(ld * _L, _L)][0] < target
                return (jnp.where(below, mid + 1, lo),
                        jnp.where(below, hi, mid))

            lo, _ = lax.fori_loop(
                0, bs_iters, step, (jnp.int32(0), jnp.int32(nvec)))
            return jnp.minimum(lo, jnp.int32(nvec))

        jv_lo = jnp.maximum(lower_bound_vec(off) - 1, 0)
        jv_hi = lower_bound_vec(off + ch)

        lane = lax.iota(jnp.int32, _L)

        def scan(j, carry):
            g = hvg_v[pl.ds(j * _L, _L)]
            iv = idx_v[pl.ds(j * _L, _L)]
            m = 1.0 / (1.0 + jnp.exp((iv - t) * (1.0 / _TEMPER)))
            inr = (g >= off) & (g < off + ch)
            loc = jnp.clip(g - off, 0, ch - 1)
            plsc.store_scatter(wbuf, [loc], m, mask=inr)
            return carry

        lax.fori_loop(jv_lo, jv_hi, scan, 0)
        pltpu.sync_copy(wbuf, w_hbm.at[pl.ds(off, ch)])

        @pl.when(cid == 0)
        def _():
            a0 = lax.div(jnp.minimum(sid * av * _L, a_last), jnp.int32(_L))

            def amin(j, carry):
                best, bpos = carry
                iv = idx_v[pl.ds(j * _L, _L)]
                d = jnp.abs(iv - t)
                pos = j * _L + lane
                upd = d < best
                return (jnp.where(upd, d, best), jnp.where(upd, pos, bpos))

            best, bpos = lax.fori_loop(
                a0, a0 + av, amin,
                (jnp.full((_L,), jnp.inf, jnp.float32),
                 jnp.zeros((_L,), jnp.int32)))
            bd_v[...] = best
            bp_v[...] = bpos
            pltpu.sync_copy(bd_v, sh_d.at[pl.ds(sid * _L, _L)])
            pltpu.sync_copy(bp_v, sh_p.at[pl.ds(sid * _L, _L)])
            plsc.subcore_barrier()

            @pl.when(sid == 0)
            def _():
                pltpu.sync_copy(sh_d, red_d)
                pltpu.sync_copy(sh_p, red_p)
                mv = red_d[pl.ds(0, _L)]
                for r in range(1, _NS):
                    mv = jnp.minimum(mv, red_d[pl.ds(r * _L, _L)])
                mn = jnp.min(mv)
                pc = jnp.full((_L,), _BIG, jnp.int32)
                for r in range(_NS):
                    pc = jnp.minimum(
                        pc, jnp.where(red_d[pl.ds(r * _L, _L)] == mn,
                                      red_p[pl.ds(r * _L, _L)], _BIG))
                p = jnp.min(pc)
                pbuf[...] = jnp.full((_L,), p, jnp.int32)
                pltpu.async_copy(unn_hbm.at[pbuf], kbuf, sem_k).wait()
                pltpu.sync_copy(kbuf, k_hbm)

    return sc_kernel


def kernel(y, eval_gene_idx, train_highly_gene_idx, train_low_gene_idx,
           index, unnorm_index, thresh):
    n_total = (eval_gene_idx.shape[0] + train_highly_gene_idx.shape[0]
               + train_low_gene_idx.shape[0])
    n_hvg = train_highly_gene_idx.shape[0]
    th16 = jnp.full((_L,), thresh, dtype=jnp.float32)
    w, kv = _build(n_total, n_hvg)(
        train_highly_gene_idx, index, unnorm_index, th16)
    return (w, w, thresh, kv[0])


# speedup vs baseline: 3.1824x; 1.0001x over previous
import functools

import jax
import jax.numpy as jnp
from jax import lax
from jax.experimental import pallas as pl
from jax.experimental.pallas import tpu as pltpu
from jax.experimental.pallas import tpu_sc as plsc

_TEMPER = 0.5
_NC = 2
_NS = 16
_L = 16
_BIG = 2 ** 30


@functools.lru_cache(maxsize=None)
def _build(n_total: int, n_hvg: int):
    nw = _NC * _NS
    ch = ((n_total + nw - 1) // nw + _L - 1) // _L * _L
    ch = max(ch, _L)
    assert ch % 8 == 0 and n_total % 8 == 0 and n_hvg % _L == 0
    last_off = n_total - ch
    nvec = n_hvg // _L
    bs_iters = max(1, (nvec - 1).bit_length()) + 1
    av = -(-n_hvg // (_NS * _L))
    a_last = n_hvg - av * _L

    mesh = plsc.VectorSubcoreMesh(core_axis_name="c", subcore_axis_name="s")

    @functools.partial(
        pl.kernel,
        out_type=(
            jax.ShapeDtypeStruct((n_total,), jnp.float32),
            jax.ShapeDtypeStruct((_L,), jnp.int32),
        ),
        mesh=mesh,
        compiler_params=pltpu.CompilerParams(
            needs_layout_passes=False,
            disable_bounds_checks=True,
            skip_device_barrier=True,
        ),
        scratch_types=[
            pltpu.VMEM((n_hvg,), jnp.int32),
            pltpu.VMEM((n_hvg,), jnp.float32),
            pltpu.VMEM((_L,), jnp.float32),
            pltpu.VMEM((ch,), jnp.float32),
            pltpu.VMEM((_L,), jnp.int32),
            pltpu.VMEM((_L,), jnp.int32),
            pltpu.VMEM((_L,), jnp.float32),
            pltpu.VMEM((_L,), jnp.int32),
            pltpu.VMEM((_NS * _L,), jnp.float32),
            pltpu.VMEM((_NS * _L,), jnp.int32),
            pltpu.VMEM_SHARED((_NS * _L,), jnp.float32),
            pltpu.VMEM_SHARED((_NS * _L,), jnp.int32),
            pltpu.SemaphoreType.DMA,
            pltpu.SemaphoreType.DMA,
            pltpu.SemaphoreType.DMA,
            pltpu.SemaphoreType.DMA,
        ],
    )
    def sc_kernel(hvg_hbm, idx_hbm, unn_hbm, th_hbm, w_hbm, k_hbm,
                  hvg_v, idx_v, th_v, wbuf, kbuf, pbuf, bd_v, bp_v,
                  red_d, red_p, sh_d, sh_p, sem_h, sem_i, sem_t, sem_k):
        cid = lax.axis_index("c")
        sid = lax.axis_index("s")
        wid = sid * _NC + cid
        off = jnp.minimum(wid * ch, last_off)

        cp_h = pltpu.async_copy(hvg_hbm, hvg_v, sem_h)
        cp_i = pltpu.async_copy(idx_hbm, idx_v, sem_i)
        cp_t = pltpu.async_copy(th_hbm, th_v, sem_t)

        ones16 = jnp.full((_L,), 1.0, jnp.float32)

        def fill(j, carry):
            wbuf[pl.ds(j * _L, _L)] = ones16
            return carry

        lax.fori_loop(0, ch // _L, fill, 0)
        cp_h.wait()
        cp_i.wait()
        cp_t.wait()
        t = th_v[...]

        def lower_bound_vec(target):
            def step(_, lohi):
                lo, hi = lohi
                mid = lax.div(lo + hi, jnp.int32(2))
                ld = jnp.minimum(mid, jnp.int32(nvec - 1))
                below = hvg_v[pl.ds(ld * _L, _L)][0] < target
                return (jnp.where(below, mid + 1, lo),
                        jnp.where(below, hi, mid))

            lo, _ = lax.fori_loop(
                0, bs_iters, step, (jnp.int32(0), jnp.int32(nvec)))
            return jnp.minimum(lo, jnp.int32(nvec))

        jv_lo = jnp.maximum(lower_bound_vec(off) - 1, 0)
        jv_hi = lower_bound_vec(off + ch)

        lane = lax.iota(jnp.int32, _L)

        def scan(j, carry):
            g = hvg_v[pl.ds(j * _L, _L)]
            iv = idx_v[pl.ds(j * _L, _L)]
            m = 1.0 / (1.0 + jnp.exp((iv - t) * (1.0 / _TEMPER)))
            inr = (g >= off) & (g < off + ch)
            loc = jnp.clip(g - off, 0, ch - 1)
            plsc.store_scatter(wbuf, [loc], m, mask=inr)
            return carry

        lax.fori_loop(jv_lo, jv_hi, scan, 0)
        pltpu.sync_copy(wbuf, w_hbm.at[pl.ds(off, ch)])

        @pl.when(cid == 0)
        def _():
            a0 = lax.div(jnp.minimum(sid * av * _L, a_last), jnp.int32(_L))

            def amin(j, carry):
                best, bpos = carry
                iv = idx_v[pl.ds(j * _L, _L)]
                d = jnp.abs(iv - t)
                pos = j * _L + lane
                upd = d < best
                return (jnp.where(upd, d, best), jnp.where(upd, pos, bpos))

            best, bpos = lax.fori_loop(
                a0, a0 + av, amin,
                (jnp.full((_L,), jnp.inf, jnp.float32),
                 jnp.zeros((_L,), jnp.int32)))
            bd_v[...] = best
            bp_v[...] = bpos
            pltpu.sync_copy(bd_v, sh_d.at[pl.ds(sid * _L, _L)])
            pltpu.sync_copy(bp_v, sh_p.at[pl.ds(sid * _L, _L)])
            plsc.subcore_barrier()

            @pl.when(sid == 0)
            def _():
                pltpu.sync_copy(sh_d, red_d)
                pltpu.sync_copy(sh_p, red_p)
                mv = red_d[pl.ds(0, _L)]
                for r in range(1, _NS):
                    mv = jnp.minimum(mv, red_d[pl.ds(r * _L, _L)])
                mn = jnp.min(mv)
                pc = jnp.full((_L,), _BIG, jnp.int32)
                for r in range(_NS):
                    pc = jnp.minimum(
                        pc, jnp.where(red_d[pl.ds(r * _L, _L)] == mn,
                                      red_p[pl.ds(r * _L, _L)], _BIG))
                p = jnp.min(pc)
                pbuf[...] = jnp.full((_L,), p, jnp.int32)
                pltpu.async_copy(unn_hbm.at[pbuf], kbuf, sem_k).wait()
                pltpu.sync_copy(kbuf, k_hbm)

    return sc_kernel


def kernel(y, eval_gene_idx, train_highly_gene_idx, train_low_gene_idx,
           index, unnorm_index, thresh):
    n_total = (eval_gene_idx.shape[0] + train_highly_gene_idx.shape[0]
               + train_low_gene_idx.shape[0])
    n_hvg = train_highly_gene_idx.shape[0]
    th16 = jnp.full((_L,), thresh, dtype=jnp.float32)
    w, kv = _build(n_total, n_hvg)(
        train_highly_gene_idx, index, unnorm_index, th16)
    return (w, w, thresh, kv[0])


# speedup vs baseline: 3.3627x; 1.0567x over previous
import functools

import jax
import jax.numpy as jnp
from jax import lax
from jax.experimental import pallas as pl
from jax.experimental.pallas import tpu as pltpu
from jax.experimental.pallas import tpu_sc as plsc

_TEMPER = 0.5
_NC = 2
_NS = 16
_L = 16
_BIG = 2 ** 30


@functools.lru_cache(maxsize=None)
def _build(n_total: int, n_hvg: int):
    nw = _NC * _NS
    ch = ((n_total + nw - 1) // nw + _L - 1) // _L * _L
    ch = max(ch, _L)
    assert ch % 8 == 0 and n_total % 8 == 0 and n_hvg % _L == 0
    last_off = n_total - ch
    nvec = n_hvg // _L
    bs_iters = max(1, (nvec - 1).bit_length()) + 1
    av = -(-n_hvg // (_NS * _L))
    a_last_v = (n_hvg - av * _L) // _L

    mesh = plsc.VectorSubcoreMesh(core_axis_name="c", subcore_axis_name="s")

    @functools.partial(
        pl.kernel,
        out_type=(
            jax.ShapeDtypeStruct((n_total,), jnp.float32),
            jax.ShapeDtypeStruct((n_total,), jnp.float32),
            jax.ShapeDtypeStruct((_L,), jnp.int32),
        ),
        mesh=mesh,
        compiler_params=pltpu.CompilerParams(
            needs_layout_passes=False,
            disable_bounds_checks=True,
            skip_device_barrier=True,
        ),
        scratch_types=[
            pltpu.VMEM((n_hvg,), jnp.int32),
            pltpu.VMEM((n_hvg,), jnp.float32),
            pltpu.VMEM((av * _L,), jnp.int32),
            pltpu.VMEM((_L,), jnp.float32),
            pltpu.VMEM((ch,), jnp.float32),
            pltpu.VMEM((_L,), jnp.int32),
            pltpu.VMEM((3 * _L,), jnp.float32),
            pltpu.VMEM((_NS * 3 * _L,), jnp.float32),
            pltpu.VMEM_SHARED((_NS * 3 * _L,), jnp.float32),
            pltpu.SemaphoreType.DMA,
            pltpu.SemaphoreType.DMA,
            pltpu.SemaphoreType.DMA,
            pltpu.SemaphoreType.DMA,
            pltpu.SemaphoreType.DMA,
            pltpu.SemaphoreType.DMA,
        ],
    )
    def sc_kernel(hvg_hbm, idx_hbm, unn_hbm, th_hbm, w_hbm, w2_hbm, k_hbm,
                  hvg_v, idx_v, unn_sl, th_v, wbuf, kbuf, stage_v, red_v,
                  sh_v, sem_h, sem_i, sem_u, sem_t, sem_w, sem_w2):
        cid = lax.axis_index("c")
        sid = lax.axis_index("s")
        wid = sid * _NC + cid
        off = jnp.minimum(wid * ch, last_off)
        a0 = jnp.minimum(sid * av, a_last_v)

        cp_h = pltpu.async_copy(hvg_hbm, hvg_v, sem_h)
        cp_i = pltpu.async_copy(idx_hbm, idx_v, sem_i)
        cp_u = pltpu.async_copy(
            unn_hbm.at[pl.ds(a0 * _L, av * _L)], unn_sl, sem_u)
        cp_t = pltpu.async_copy(th_hbm, th_v, sem_t)

        ones16 = jnp.full((_L,), 1.0, jnp.float32)
        for j in range(ch // _L):
            wbuf[pl.ds(j * _L, _L)] = ones16

        cp_t.wait()
        cp_i.wait()
        cp_u.wait()
        t = th_v[...]
        lane = lax.iota(jnp.int32, _L)

        @pl.when(cid == 0)
        def _():
            def amin(j, carry):
                best, bpos, buvl = carry
                iv = idx_v[pl.ds(j * _L, _L)]
                uv = unn_sl[pl.ds((j - a0) * _L, _L)]
                d = jnp.abs(iv - t)
                pos = j * _L + lane
                upd = d < best
                return (jnp.where(upd, d, best), jnp.where(upd, pos, bpos),
                        jnp.where(upd, uv, buvl))

            best, bpos, buvl = lax.fori_loop(
                a0, a0 + av, amin,
                (jnp.full((_L,), jnp.inf, jnp.float32),
                 jnp.zeros((_L,), jnp.int32),
                 jnp.zeros((_L,), jnp.int32)))
            stage_v[pl.ds(0, _L)] = best
            stage_v[pl.ds(_L, _L)] = plsc.bitcast(bpos, jnp.float32)
            stage_v[pl.ds(2 * _L, _L)] = plsc.bitcast(buvl, jnp.float32)
            pltpu.sync_copy(stage_v, sh_v.at[pl.ds(sid * 3 * _L, 3 * _L)])
            plsc.subcore_barrier()

            @pl.when(sid == 0)
            def _():
                pltpu.sync_copy(sh_v, red_v)
                rows = []
                for r in range(_NS):
                    dr = red_v[pl.ds(r * 3 * _L, _L)]
                    pr = plsc.bitcast(red_v[pl.ds(r * 3 * _L + _L, _L)],
                                      jnp.int32)
                    ur = plsc.bitcast(red_v[pl.ds(r * 3 * _L + 2 * _L, _L)],
                                      jnp.int32)
                    rows.append((dr, pr, ur))
                mv = rows[0][0]
                for dr, _, _ in rows[1:]:
                    mv = jnp.minimum(mv, dr)
                mn = jnp.min(mv)
                pc = jnp.full((_L,), _BIG, jnp.int32)
                for dr, pr, _ in rows:
                    pc = jnp.minimum(pc, jnp.where(dr == mn, pr, _BIG))
                p = jnp.min(pc)
                kc = jnp.full((_L,), _BIG, jnp.int32)
                for dr, pr, ur in rows:
                    kc = jnp.minimum(
                        kc, jnp.where((dr == mn) & (pr == p), ur, _BIG))
                kbuf[...] = jnp.full((_L,), jnp.min(kc), jnp.int32)
                pltpu.sync_copy(kbuf, k_hbm)

        cp_h.wait()

        def lower_bound_vec(target):
            def step(_, lohi):
                lo, hi = lohi
                mid = lax.div(lo + hi, jnp.int32(2))
                ld = jnp.minimum(mid, jnp.int32(nvec - 1))
                below = hvg_v[pl.ds(ld * _L, _L)][0] < target
                return (jnp.where(below, mid + 1, lo),
                        jnp.where(below, hi, mid))

            lo, _ = lax.fori_loop(
                0, bs_iters, step, (jnp.int32(0), jnp.int32(nvec)))
            return jnp.minimum(lo, jnp.int32(nvec))

        jv_lo = jnp.maximum(lower_bound_vec(off) - 1, 0)
        jv_hi = lower_bound_vec(off + ch)

        def scan(j, carry):
            g = hvg_v[pl.ds(j * _L, _L)]
            iv = idx_v[pl.ds(j * _L, _L)]
            m = 1.0 / (1.0 + jnp.exp((iv - t) * (1.0 / _TEMPER)))
            inr = (g >= off) & (g < off + ch)
            loc = jnp.clip(g - off, 0, ch - 1)
            plsc.store_scatter(wbuf, [loc], m, mask=inr)
            return carry

        lax.fori_loop(jv_lo, jv_hi, scan, 0)
        cp_w = pltpu.async_copy(wbuf, w_hbm.at[pl.ds(off, ch)], sem_w)
        cp_w2 = pltpu.async_copy(wbuf, w2_hbm.at[pl.ds(off, ch)], sem_w2)
        cp_w.wait()
        cp_w2.wait()

    return sc_kernel


def kernel(y, eval_gene_idx, train_highly_gene_idx, train_low_gene_idx,
           index, unnorm_index, thresh):
    n_total = (eval_gene_idx.shape[0] + train_highly_gene_idx.shape[0]
               + train_low_gene_idx.shape[0])
    n_hvg = train_highly_gene_idx.shape[0]
    th16 = jnp.full((_L,), thresh, dtype=jnp.float32)
    w, w2, kv = _build(n_total, n_hvg)(
        train_highly_gene_idx, index, unnorm_index, th16)
    return (w, w2, thresh, kv[0])


# speedup vs baseline: 3.3983x; 1.0106x over previous
import functools

import jax
import jax.numpy as jnp
from jax import lax
from jax.experimental import pallas as pl
from jax.experimental.pallas import tpu as pltpu
from jax.experimental.pallas import tpu_sc as plsc

_TEMPER = 0.5
_NC = 2
_NS = 16
_L = 16
_BIG = 2 ** 30


@functools.lru_cache(maxsize=None)
def _build(n_total: int, n_hvg: int):
    nwk = _NC * _NS - 1
    ch = ((n_total + nwk - 1) // nwk + _L - 1) // _L * _L
    ch = max(ch, _L)
    assert ch % 8 == 0 and n_total % 8 == 0 and n_hvg % _L == 0
    last_off = n_total - ch
    nvec = n_hvg // _L
    bs_iters = max(1, (nvec - 1).bit_length()) + 1
    wcap = ch + 2 * _L
    wb_max = (n_hvg - wcap) // _L
    assert wb_max >= 0 and (n_hvg - wcap) % _L == 0
    av = -(-n_hvg // (_NS * _L))
    a_last_v = (n_hvg - av * _L) // _L

    mesh = plsc.VectorSubcoreMesh(core_axis_name="c", subcore_axis_name="s")

    @functools.partial(
        pl.kernel,
        out_type=(
            jax.ShapeDtypeStruct((n_total,), jnp.float32),
            jax.ShapeDtypeStruct((n_total,), jnp.float32),
            jax.ShapeDtypeStruct((_L,), jnp.int32),
        ),
        mesh=mesh,
        compiler_params=pltpu.CompilerParams(
            needs_layout_passes=False,
            disable_bounds_checks=True,
            skip_device_barrier=True,
        ),
        scratch_types=[
            pltpu.VMEM((n_hvg,), jnp.int32),
            pltpu.VMEM((wcap,), jnp.float32),
            pltpu.VMEM((av * _L,), jnp.float32),
            pltpu.VMEM((av * _L,), jnp.int32),
            pltpu.VMEM((_L,), jnp.float32),
            pltpu.VMEM((ch,), jnp.float32),
            pltpu.VMEM((_L,), jnp.int32),
            pltpu.VMEM((3 * _L,), jnp.float32),
            pltpu.VMEM((_NS * 3 * _L,), jnp.float32),
            pltpu.VMEM_SHARED((_NS * 3 * _L,), jnp.float32),
            pltpu.SemaphoreType.DMA,
            pltpu.SemaphoreType.DMA,
            pltpu.SemaphoreType.DMA,
            pltpu.SemaphoreType.DMA,
            pltpu.SemaphoreType.DMA,
            pltpu.SemaphoreType.DMA,
        ],
    )
    def sc_kernel(hvg_hbm, idx_hbm, unn_hbm, th_hbm, w_hbm, w2_hbm, k_hbm,
                  hvg_v, idx_w, idx_am, unn_sl, th_v, wbuf, kbuf, stage_v,
                  red_v, sh_v, sem_h, sem_ia, sem_u, sem_t, sem_w, sem_w2):
        cid = lax.axis_index("c")
        sid = lax.axis_index("s")
        wid = sid * _NC + cid
        off = jnp.minimum(jnp.maximum(wid - 1, 0) * ch, last_off)
        a0 = jnp.minimum(sid * av, a_last_v)

        cp_t = pltpu.async_copy(th_hbm, th_v.at[pl.ds(0, 1)], sem_t)
        cp_ia = pltpu.async_copy(
            idx_hbm.at[pl.ds(a0 * _L, av * _L)], idx_am, sem_ia)
        cp_u = pltpu.async_copy(
            unn_hbm.at[pl.ds(a0 * _L, av * _L)], unn_sl, sem_u)
        cp_h = pltpu.async_copy(hvg_hbm, hvg_v, sem_h)

        ones16 = jnp.full((_L,), 1.0, jnp.float32)
        for j in range(ch // _L):
            wbuf[pl.ds(j * _L, _L)] = ones16

        cp_t.wait()
        cp_ia.wait()
        cp_u.wait()
        t = jnp.full((_L,), th_v[...][0], jnp.float32)
        lane = lax.iota(jnp.int32, _L)

        @pl.when(cid == 0)
        def _():
            def amin(j, carry):
                best, bpos, buvl = carry
                iv = idx_am[pl.ds(j * _L, _L)]
                uv = unn_sl[pl.ds(j * _L, _L)]
                d = jnp.abs(iv - t)
                pos = (a0 + j) * _L + lane
                upd = d < best
                return (jnp.where(upd, d, best), jnp.where(upd, pos, bpos),
                        jnp.where(upd, uv, buvl))

            best, bpos, buvl = lax.fori_loop(
                0, av, amin,
                (jnp.full((_L,), jnp.inf, jnp.float32),
                 jnp.zeros((_L,), jnp.int32),
                 jnp.zeros((_L,), jnp.int32)))
            stage_v[pl.ds(0, _L)] = best
            stage_v[pl.ds(_L, _L)] = plsc.bitcast(bpos, jnp.float32)
            stage_v[pl.ds(2 * _L, _L)] = plsc.bitcast(buvl, jnp.float32)
            pltpu.sync_copy(stage_v, sh_v.at[pl.ds(sid * 3 * _L, 3 * _L)])
            plsc.subcore_barrier()

            @pl.when(sid == 0)
            def _():
                pltpu.sync_copy(sh_v, red_v)
                rows = []
                for r in range(_NS):
                    dr = red_v[pl.ds(r * 3 * _L, _L)]
                    pr = plsc.bitcast(red_v[pl.ds(r * 3 * _L + _L, _L)],
                                      jnp.int32)
                    ur = plsc.bitcast(red_v[pl.ds(r * 3 * _L + 2 * _L, _L)],
                                      jnp.int32)
                    rows.append((dr, pr, ur))
                mv = rows[0][0]
                for dr, _, _ in rows[1:]:
                    mv = jnp.minimum(mv, dr)
                mn = jnp.min(mv)
                pc = jnp.full((_L,), _BIG, jnp.int32)
                for dr, pr, _ in rows:
                    pc = jnp.minimum(pc, jnp.where(dr == mn, pr, _BIG))
                p = jnp.min(pc)
                kc = jnp.full((_L,), _BIG, jnp.int32)
                for dr, pr, ur in rows:
                    kc = jnp.minimum(
                        kc, jnp.where((dr == mn) & (pr == p), ur, _BIG))
                kbuf[...] = jnp.full((_L,), jnp.min(kc), jnp.int32)
                pltpu.sync_copy(kbuf, k_hbm)

        cp_h.wait()

        @pl.when(wid > 0)
        def _():
            def lower_bound_vec(target):
                def step(_, lohi):
                    lo, hi = lohi
                    mid = lax.div(lo + hi, jnp.int32(2))
                    ld = jnp.minimum(mid, jnp.int32(nvec - 1))
                    below = hvg_v[pl.ds(ld * _L, _L)][0] < target
                    return (jnp.where(below, mid + 1, lo),
                            jnp.where(below, hi, mid))

                lo, _ = lax.fori_loop(
                    0, bs_iters, step, (jnp.int32(0), jnp.int32(nvec)))
                return jnp.minimum(lo, jnp.int32(nvec))

            jv_lo = jnp.maximum(lower_bound_vec(off) - 1, 0)
            jv_hi = lower_bound_vec(off + ch)
            wb = jnp.minimum(jv_lo, jnp.int32(wb_max))
            pltpu.sync_copy(idx_hbm.at[pl.ds(wb * _L, wcap)], idx_w)

            def scan(j, carry):
                g = hvg_v[pl.ds(j * _L, _L)]
                iv = idx_w[pl.ds((j - wb) * _L, _L)]
                m = 1.0 / (1.0 + jnp.exp((iv - t) * (1.0 / _TEMPER)))
                inr = (g >= off) & (g < off + ch)
                loc = jnp.clip(g - off, 0, ch - 1)
                plsc.store_scatter(wbuf, [loc], m, mask=inr)
                return carry

            lax.fori_loop(jv_lo, jv_hi, scan, 0)
            cp_w = pltpu.async_copy(wbuf, w_hbm.at[pl.ds(off, ch)], sem_w)
            cp_w2 = pltpu.async_copy(wbuf, w2_hbm.at[pl.ds(off, ch)], sem_w2)
            cp_w.wait()
            cp_w2.wait()

    return sc_kernel


def kernel(y, eval_gene_idx, train_highly_gene_idx, train_low_gene_idx,
           index, unnorm_index, thresh):
    n_total = (eval_gene_idx.shape[0] + train_highly_gene_idx.shape[0]
               + train_low_gene_idx.shape[0])
    n_hvg = train_highly_gene_idx.shape[0]
    w, w2, kv = _build(n_total, n_hvg)(
        train_highly_gene_idx, index, unnorm_index, thresh.reshape(1))
    return (w, w2, thresh, kv[0])


# speedup vs baseline: 3.8848x; 1.1432x over previous
import functools

import jax
import jax.numpy as jnp
from jax import lax
from jax.experimental import pallas as pl
from jax.experimental.pallas import tpu as pltpu
from jax.experimental.pallas import tpu_sc as plsc

_TEMPER = 0.5
_NC = 2
_NS = 16
_L = 16
_BIG = 2 ** 30


@functools.lru_cache(maxsize=None)
def _build(n_total: int, n_hvg: int):
    nwk = _NC * _NS - 1
    ch = ((n_total + nwk - 1) // nwk + _L - 1) // _L * _L
    ch = max(ch, _L)
    assert ch % 8 == 0 and n_total % 8 == 0 and n_hvg % _L == 0
    last_off = n_total - ch
    nvec = n_hvg // _L
    dps = -(-nvec // _NS)
    dps = ((dps + _L - 1) // _L) * _L
    sl_cap = dps * _L + _L
    sl_last = n_hvg - sl_cap
    assert sl_last >= 0 and sl_last % _L == 0 and _NS * dps >= nvec
    bs_iters = max(1, (nvec - 1).bit_length()) + 1
    wcap = ch + 2 * _L
    wb_max = (n_hvg - wcap) // _L
    assert wb_max >= 0 and (n_hvg - wcap) % _L == 0
    av = -(-n_hvg // (_NS * _L))
    a_last_v = (n_hvg - av * _L) // _L

    mesh = plsc.VectorSubcoreMesh(core_axis_name="c", subcore_axis_name="s")

    @functools.partial(
        pl.kernel,
        out_type=(
            jax.ShapeDtypeStruct((n_total,), jnp.float32),
            jax.ShapeDtypeStruct((n_total,), jnp.float32),
            jax.ShapeDtypeStruct((_L,), jnp.int32),
        ),
        mesh=mesh,
        compiler_params=pltpu.CompilerParams(
            needs_layout_passes=False,
            disable_bounds_checks=True,
            skip_device_barrier=True,
        ),
        scratch_types=[
            pltpu.VMEM((sl_cap,), jnp.int32),
            pltpu.VMEM((dps,), jnp.int32),
            pltpu.VMEM((_NS * dps,), jnp.int32),
            pltpu.VMEM((wcap,), jnp.int32),
            pltpu.VMEM((wcap,), jnp.float32),
            pltpu.VMEM((av * _L,), jnp.float32),
            pltpu.VMEM((av * _L,), jnp.int32),
            pltpu.VMEM((_L,), jnp.float32),
            pltpu.VMEM((ch,), jnp.float32),
            pltpu.VMEM((_L,), jnp.int32),
            pltpu.VMEM((3 * _L,), jnp.float32),
            pltpu.VMEM((_NS * 3 * _L,), jnp.float32),
            pltpu.VMEM_SHARED((_NS * dps,), jnp.int32),
            pltpu.VMEM_SHARED((_NS * 3 * _L,), jnp.float32),
            pltpu.SemaphoreType.DMA,
            pltpu.SemaphoreType.DMA,
            pltpu.SemaphoreType.DMA,
            pltpu.SemaphoreType.DMA,
            pltpu.SemaphoreType.DMA,
            pltpu.SemaphoreType.DMA,
            pltpu.SemaphoreType.DMA,
        ],
    )
    def sc_kernel(hvg_hbm, idx_hbm, unn_hbm, th_hbm, w_hbm, w2_hbm, k_hbm,
                  sl_v, dirc_v, dir_v, hvg_w, idx_w, idx_am, unn_sl, th_v,
                  wbuf, kbuf, stage_v, red_v, dir_sh, sh_v,
                  sem_sl, sem_ia, sem_u, sem_t, sem_hw, sem_w, sem_w2):
        cid = lax.axis_index("c")
        sid = lax.axis_index("s")
        wid = sid * _NC + cid
        off = jnp.minimum(jnp.maximum(wid - 1, 0) * ch, last_off)
        a0 = jnp.minimum(sid * av, a_last_v)
        sl_base = jnp.minimum(sid * dps * _L, jnp.int32(sl_last))
        dshift = sid * dps - lax.div(sl_base, jnp.int32(_L))

        cp_sl = pltpu.async_copy(
            hvg_hbm.at[pl.ds(sl_base, sl_cap)], sl_v, sem_sl)
        cp_t = pltpu.async_copy(th_hbm, th_v.at[pl.ds(0, 1)], sem_t)
        cp_ia = pltpu.async_copy(
            idx_hbm.at[pl.ds(a0 * _L, av * _L)], idx_am, sem_ia)
        cp_u = pltpu.async_copy(
            unn_hbm.at[pl.ds(a0 * _L, av * _L)], unn_sl, sem_u)

        ones16 = jnp.full((_L,), 1.0, jnp.float32)
        for j in range(ch // _L):
            wbuf[pl.ds(j * _L, _L)] = ones16

        lane = lax.iota(jnp.int32, _L)

        cp_sl.wait()
        for g in range(dps // _L):
            e = jnp.minimum((g * _L + dshift) * _L + lane * _L,
                            jnp.int32(sl_cap - _L))
            dirc_v[pl.ds(g * _L, _L)] = plsc.load_gather(sl_v, [e])
        pltpu.sync_copy(dirc_v, dir_sh.at[pl.ds(sid * dps, dps)])
        plsc.subcore_barrier()
        pltpu.sync_copy(dir_sh, dir_v)

        cp_t.wait()
        cp_ia.wait()
        cp_u.wait()
        t = jnp.full((_L,), th_v[...][0], jnp.float32)

        @pl.when(cid == 0)
        def _():
            def amin(j, carry):
                best, bpos, buvl = carry
                iv = idx_am[pl.ds(j * _L, _L)]
                uv = unn_sl[pl.ds(j * _L, _L)]
                d = jnp.abs(iv - t)
                pos = (a0 + j) * _L + lane
                upd = d < best
                return (jnp.where(upd, d, best), jnp.where(upd, pos, bpos),
                        jnp.where(upd, uv, buvl))

            best, bpos, buvl = lax.fori_loop(
                0, av, amin,
                (jnp.full((_L,), jnp.inf, jnp.float32),
                 jnp.zeros((_L,), jnp.int32),
                 jnp.zeros((_L,), jnp.int32)))
            stage_v[pl.ds(0, _L)] = best
            stage_v[pl.ds(_L, _L)] = plsc.bitcast(bpos, jnp.float32)
            stage_v[pl.ds(2 * _L, _L)] = plsc.bitcast(buvl, jnp.float32)
            pltpu.sync_copy(stage_v, sh_v.at[pl.ds(sid * 3 * _L, 3 * _L)])
            plsc.subcore_barrier()

            @pl.when(sid == 0)
            def _():
                pltpu.sync_copy(sh_v, red_v)
                rows = []
                for r in range(_NS):
                    dr = red_v[pl.ds(r * 3 * _L, _L)]
                    pr = plsc.bitcast(red_v[pl.ds(r * 3 * _L + _L, _L)],
                                      jnp.int32)
                    ur = plsc.bitcast(red_v[pl.ds(r * 3 * _L + 2 * _L, _L)],
                                      jnp.int32)
                    rows.append((dr, pr, ur))
                mv = rows[0][0]
                for dr, _, _ in rows[1:]:
                    mv = jnp.minimum(mv, dr)
                mn = jnp.min(mv)
                pc = jnp.full((_L,), _BIG, jnp.int32)
                for dr, pr, _ in rows:
                    pc = jnp.minimum(pc, jnp.where(dr == mn, pr, _BIG))
                p = jnp.min(pc)
                kc = jnp.full((_L,), _BIG, jnp.int32)
                for dr, pr, ur in rows:
                    kc = jnp.minimum(
                        kc, jnp.where((dr == mn) & (pr == p), ur, _BIG))
                kbuf[...] = jnp.full((_L,), jnp.min(kc), jnp.int32)
                pltpu.sync_copy(kbuf, k_hbm)

        @pl.when(wid > 0)
        def _():
            def lower_bound_vec(target):
                def step(_, lohi):
                    lo, hi = lohi
                    mid = lax.div(lo + hi, jnp.int32(2))
                    ld = jnp.minimum(mid, jnp.int32(nvec - 1))
                    v = plsc.load_gather(
                        dir_v, [jnp.full((_L,), ld, jnp.int32)])[0]
                    below = v < target
                    return (jnp.where(below, mid + 1, lo),
                            jnp.where(below, hi, mid))

                lo, _ = lax.fori_loop(
                    0, bs_iters, step, (jnp.int32(0), jnp.int32(nvec)))
                return jnp.minimum(lo, jnp.int32(nvec))

            jv_lo = jnp.maximum(lower_bound_vec(off) - 1, 0)
            jv_hi = lower_bound_vec(off + ch)
            wb = jnp.minimum(jv_lo, jnp.int32(wb_max))
            cp_hw = pltpu.async_copy(
                hvg_hbm.at[pl.ds(wb * _L, wcap)], hvg_w, sem_hw)
            cp_iw = pltpu.async_copy(
                idx_hbm.at[pl.ds(wb * _L, wcap)], idx_w, sem_sl)
            cp_hw.wait()
            cp_iw.wait()

            def scan(j, carry):
                g = hvg_w[pl.ds((j - wb) * _L, _L)]
                iv = idx_w[pl.ds((j - wb) * _L, _L)]
                m = 1.0 / (1.0 + jnp.exp((iv - t) * (1.0 / _TEMPER)))
                inr = (g >= off) & (g < off + ch)
                loc = jnp.clip(g - off, 0, ch - 1)
                plsc.store_scatter(wbuf, [loc], m, mask=inr)
                return carry

            lax.fori_loop(jv_lo, jv_hi, scan, 0)
            cp_w = pltpu.async_copy(wbuf, w_hbm.at[pl.ds(off, ch)], sem_w)
            cp_w2 = pltpu.async_copy(wbuf, w2_hbm.at[pl.ds(off, ch)], sem_w2)
            cp_w.wait()
            cp_w2.wait()

    return sc_kernel


def kernel(y, eval_gene_idx, train_highly_gene_idx, train_low_gene_idx,
           index, unnorm_index, thresh):
    n_total = (eval_gene_idx.shape[0] + train_highly_gene_idx.shape[0]
               + train_low_gene_idx.shape[0])
    n_hvg = train_highly_gene_idx.shape[0]
    w, w2, kv = _build(n_total, n_hvg)(
        train_highly_gene_idx, index, unnorm_index, thresh.reshape(1))
    return (w, w2, thresh, kv[0])


# speedup vs baseline: 3.8896x; 1.0012x over previous
import functools

import jax
import jax.numpy as jnp
from jax import lax
from jax.experimental import pallas as pl
from jax.experimental.pallas import tpu as pltpu
from jax.experimental.pallas import tpu_sc as plsc

_TEMPER = 0.5
_NC = 2
_NS = 16
_L = 16
_BIG = 2 ** 30


@functools.lru_cache(maxsize=None)
def _build(n_total: int, n_hvg: int):
    nwk = _NC * _NS - 1
    ch = ((n_total + nwk - 1) // nwk + _L - 1) // _L * _L
    ch = max(ch, _L)
    assert ch % 8 == 0 and n_total % 8 == 0 and n_hvg % _L == 0
    last_off = n_total - ch
    nvec = n_hvg // _L
    dps = -(-nvec // _NS)
    dps = ((dps + _L - 1) // _L) * _L
    sl_cap = dps * _L + _L
    sl_last = n_hvg - sl_cap
    assert sl_last >= 0 and sl_last % _L == 0 and _NS * dps >= nvec
    bs_iters = max(1, (nvec - 1).bit_length()) + 1
    wcap = ch + 2 * _L
    wb_max = (n_hvg - wcap) // _L
    assert wb_max >= 0 and (n_hvg - wcap) % _L == 0
    av = -(-n_hvg // (_NS * _L))
    a_last_v = (n_hvg - av * _L) // _L

    mesh = plsc.VectorSubcoreMesh(core_axis_name="c", subcore_axis_name="s")

    @functools.partial(
        pl.kernel,
        out_type=(
            jax.ShapeDtypeStruct((n_total,), jnp.float32),
            jax.ShapeDtypeStruct((n_total,), jnp.float32),
            jax.ShapeDtypeStruct((_L,), jnp.int32),
        ),
        mesh=mesh,
        compiler_params=pltpu.CompilerParams(
            needs_layout_passes=False,
            disable_bounds_checks=True,
            skip_device_barrier=True,
        ),
        scratch_types=[
            pltpu.VMEM((sl_cap,), jnp.int32),
            pltpu.VMEM((dps,), jnp.int32),
            pltpu.VMEM((_NS * dps,), jnp.int32),
            pltpu.VMEM((wcap,), jnp.int32),
            pltpu.VMEM((wcap,), jnp.float32),
            pltpu.VMEM((av * _L,), jnp.float32),
            pltpu.VMEM((av * _L,), jnp.int32),
            pltpu.VMEM((_L,), jnp.float32),
            pltpu.VMEM((ch,), jnp.float32),
            pltpu.VMEM((_L,), jnp.int32),
            pltpu.VMEM((3 * _L,), jnp.float32),
            pltpu.VMEM((_NS * 3 * _L,), jnp.float32),
            pltpu.VMEM_SHARED((_NS * dps,), jnp.int32),
            pltpu.VMEM_SHARED((_NS * 3 * _L,), jnp.float32),
            pltpu.SemaphoreType.DMA,
            pltpu.SemaphoreType.DMA,
            pltpu.SemaphoreType.DMA,
            pltpu.SemaphoreType.DMA,
            pltpu.SemaphoreType.DMA,
            pltpu.SemaphoreType.DMA,
            pltpu.SemaphoreType.DMA,
        ],
    )
    def sc_kernel(hvg_hbm, idx_hbm, unn_hbm, th_hbm, w_hbm, w2_hbm, k_hbm,
                  sl_v, dirc_v, dir_v, hvg_w, idx_w, idx_am, unn_sl, th_v,
                  wbuf, kbuf, stage_v, red_v, dir_sh, sh_v,
                  sem_sl, sem_ia, sem_u, sem_t, sem_hw, sem_w, sem_w2):
        cid = lax.axis_index("c")
        sid = lax.axis_index("s")
        wid = sid * _NC + cid
        off = jnp.minimum(jnp.maximum(wid - 1, 0) * ch, last_off)
        a0 = jnp.minimum(sid * av, a_last_v)
        sl_base = jnp.minimum(sid * dps * _L, jnp.int32(sl_last))
        dshift = sid * dps - lax.div(sl_base, jnp.int32(_L))

        cp_sl = pltpu.async_copy(
            hvg_hbm.at[pl.ds(sl_base, sl_cap)], sl_v, sem_sl)
        cp_t = pltpu.async_copy(th_hbm, th_v.at[pl.ds(0, 1)], sem_t)
        cp_ia = pltpu.async_copy(
            idx_hbm.at[pl.ds(a0 * _L, av * _L)], idx_am, sem_ia)
        cp_u = pltpu.async_copy(
            unn_hbm.at[pl.ds(a0 * _L, av * _L)], unn_sl, sem_u)

        ones16 = jnp.full((_L,), 1.0, jnp.float32)
        for j in range(ch // _L):
            wbuf[pl.ds(j * _L, _L)] = ones16

        lane = lax.iota(jnp.int32, _L)

        cp_sl.wait()
        for g in range(dps // _L):
            e = jnp.minimum((g * _L + dshift) * _L + lane * _L,
                            jnp.int32(sl_cap - _L))
            dirc_v[pl.ds(g * _L, _L)] = plsc.load_gather(sl_v, [e])
        pltpu.sync_copy(dirc_v, dir_sh.at[pl.ds(sid * dps, dps)])

        cp_t.wait()
        cp_ia.wait()
        cp_u.wait()
        t = jnp.full((_L,), th_v[...][0], jnp.float32)

        @pl.when(cid == 0)
        def _():
            def amin(j, carry):
                best, bpos, buvl = carry
                iv = idx_am[pl.ds(j * _L, _L)]
                uv = unn_sl[pl.ds(j * _L, _L)]
                d = jnp.abs(iv - t)
                pos = (a0 + j) * _L + lane
                upd = d < best
                return (jnp.where(upd, d, best), jnp.where(upd, pos, bpos),
                        jnp.where(upd, uv, buvl))

            best, bpos, buvl = lax.fori_loop(
                0, av, amin,
                (jnp.full((_L,), jnp.inf, jnp.float32),
                 jnp.zeros((_L,), jnp.int32),
                 jnp.zeros((_L,), jnp.int32)))
            stage_v[pl.ds(0, _L)] = best
            stage_v[pl.ds(_L, _L)] = plsc.bitcast(bpos, jnp.float32)
            stage_v[pl.ds(2 * _L, _L)] = plsc.bitcast(buvl, jnp.float32)
            pltpu.sync_copy(stage_v, sh_v.at[pl.ds(sid * 3 * _L, 3 * _L)])

        plsc.subcore_barrier()

        @pl.when(wid == 0)
        def _():
                pltpu.sync_copy(sh_v, red_v)
                rows = []
                for r in range(_NS):
                    dr = red_v[pl.ds(r * 3 * _L, _L)]
                    pr = plsc.bitcast(red_v[pl.ds(r * 3 * _L + _L, _L)],
                                      jnp.int32)
                    ur = plsc.bitcast(red_v[pl.ds(r * 3 * _L + 2 * _L, _L)],
                                      jnp.int32)
                    rows.append((dr, pr, ur))
                mv = rows[0][0]
                for dr, _, _ in rows[1:]:
                    mv = jnp.minimum(mv, dr)
                mn = jnp.min(mv)
                pc = jnp.full((_L,), _BIG, jnp.int32)
                for dr, pr, _ in rows:
                    pc = jnp.minimum(pc, jnp.where(dr == mn, pr, _BIG))
                p = jnp.min(pc)
                kc = jnp.full((_L,), _BIG, jnp.int32)
                for dr, pr, ur in rows:
                    kc = jnp.minimum(
                        kc, jnp.where((dr == mn) & (pr == p), ur, _BIG))
                kbuf[...] = jnp.full((_L,), jnp.min(kc), jnp.int32)
                pltpu.sync_copy(kbuf, k_hbm)

        @pl.when(wid > 0)
        def _():
            pltpu.sync_copy(dir_sh, dir_v)

            def lower_bound_vec(target):
                def step(_, lohi):
                    lo, hi = lohi
                    mid = lax.div(lo + hi, jnp.int32(2))
                    ld = jnp.minimum(mid, jnp.int32(nvec - 1))
                    v = plsc.load_gather(
                        dir_v, [jnp.full((_L,), ld, jnp.int32)])[0]
                    below = v < target
                    return (jnp.where(below, mid + 1, lo),
                            jnp.where(below, hi, mid))

                lo, _ = lax.fori_loop(
                    0, bs_iters, step, (jnp.int32(0), jnp.int32(nvec)))
                return jnp.minimum(lo, jnp.int32(nvec))

            jv_lo = jnp.maximum(lower_bound_vec(off) - 1, 0)
            jv_hi = lower_bound_vec(off + ch)
            wb = jnp.minimum(jv_lo, jnp.int32(wb_max))
            cp_hw = pltpu.async_copy(
                hvg_hbm.at[pl.ds(wb * _L, wcap)], hvg_w, sem_hw)
            cp_iw = pltpu.async_copy(
                idx_hbm.at[pl.ds(wb * _L, wcap)], idx_w, sem_sl)
            cp_hw.wait()
            cp_iw.wait()

            def scan(j, carry):
                g = hvg_w[pl.ds((j - wb) * _L, _L)]
                iv = idx_w[pl.ds((j - wb) * _L, _L)]
                m = 1.0 / (1.0 + jnp.exp((iv - t) * (1.0 / _TEMPER)))
                inr = (g >= off) & (g < off + ch)
                loc = jnp.clip(g - off, 0, ch - 1)
                plsc.store_scatter(wbuf, [loc], m, mask=inr)
                return carry

            lax.fori_loop(jv_lo, jv_hi, scan, 0)
            cp_w = pltpu.async_copy(wbuf, w_hbm.at[pl.ds(off, ch)], sem_w)
            cp_w2 = pltpu.async_copy(wbuf, w2_hbm.at[pl.ds(off, ch)], sem_w2)
            cp_w.wait()
            cp_w2.wait()

    return sc_kernel


def kernel(y, eval_gene_idx, train_highly_gene_idx, train_low_gene_idx,
           index, unnorm_index, thresh):
    n_total = (eval_gene_idx.shape[0] + train_highly_gene_idx.shape[0]
               + train_low_gene_idx.shape[0])
    n_hvg = train_highly_gene_idx.shape[0]
    w, w2, kv = _build(n_total, n_hvg)(
        train_highly_gene_idx, index, unnorm_index, thresh.reshape(1))
    return (w, w2, thresh, kv[0])
